# baseline (speedup 1.0000x reference)
import functools

import jax
import jax.numpy as jnp
from jax.experimental import pallas as pl

K_CLS = 7
NSAMPLE = 32
BASE_RADIUS = 0.15
RADIUS_SCALING = 1.5


def _stem_body(x_ref, w_ref, b_ref, o_ref):
    o_ref[...] = jnp.maximum(
        jnp.dot(x_ref[...], w_ref[...], preferred_element_type=jnp.float32)
        + b_ref[...], 0.0)


def _stem(points2d, W, b):
    R = points2d.shape[0]
    bm = 4096
    return pl.pallas_call(
        _stem_body,
        grid=(R // bm,),
        in_specs=[
            pl.BlockSpec((bm, 4), lambda i: (i, 0)),
            pl.BlockSpec((4, 32), lambda i: (0, 0)),
            pl.BlockSpec((1, 32), lambda i: (0, 0)),
        ],
        out_specs=pl.BlockSpec((bm, 32), lambda i: (i, 0)),
        out_shape=jax.ShapeDtypeStruct((R, 32), jnp.float32),
    )(points2d, W, b[None])


def _sa_body(S, bm, g_ref, fc_ref, w1_ref, b1_ref, w2_ref, b2_ref,
             wr_ref, br_ref, o_ref):
    x = g_ref[0]
    h = jnp.maximum(
        jnp.dot(x, w1_ref[...], preferred_element_type=jnp.float32)
        + b1_ref[...], 0.0)
    h = jnp.dot(h, w2_ref[...], preferred_element_type=jnp.float32) \
        + b2_ref[...]
    Co = h.shape[-1]
    h = h.reshape(bm, S, Co).max(axis=1)
    r = jnp.dot(fc_ref[0], wr_ref[...],
                preferred_element_type=jnp.float32) + br_ref[...]
    o_ref[0] = jnp.maximum(h + r, 0.0)


def _sa_mlp(g2, fc, W1, b1, W2, b2, Wr, br):
    B, MS, Ci = g2.shape
    M = fc.shape[1]
    Cin = fc.shape[2]
    S = MS // M
    Co = W1.shape[1]
    bm = min(M, 256)
    body = functools.partial(_sa_body, S, bm)
    return pl.pallas_call(
        body,
        grid=(B, M // bm),
        in_specs=[
            pl.BlockSpec((1, bm * S, Ci), lambda b, m: (b, m, 0)),
            pl.BlockSpec((1, bm, Cin), lambda b, m: (b, m, 0)),
            pl.BlockSpec((Ci, Co), lambda b, m: (0, 0)),
            pl.BlockSpec((1, Co), lambda b, m: (0, 0)),
            pl.BlockSpec((Co, Co), lambda b, m: (0, 0)),
            pl.BlockSpec((1, Co), lambda b, m: (0, 0)),
            pl.BlockSpec((Cin, Co), lambda b, m: (0, 0)),
            pl.BlockSpec((1, Co), lambda b, m: (0, 0)),
        ],
        out_specs=pl.BlockSpec((1, bm, Co), lambda b, m: (b, m, 0)),
        out_shape=jax.ShapeDtypeStruct((B, M, Co), jnp.float32),
    )(g2, fc, W1, b1[None], W2, b2[None], Wr, br[None])


def _ln_head(x, w1, b1, lw, lb, w2, b2):
    h = jnp.dot(x, w1, preferred_element_type=jnp.float32) + b1
    mu = jnp.mean(h, axis=-1, keepdims=True)
    var = jnp.mean((h - mu) ** 2, axis=-1, keepdims=True)
    h = (h - mu) * jax.lax.rsqrt(var + 1e-5) * lw + lb
    h = jnp.maximum(h, 0.0)
    return jnp.dot(h, w2, preferred_element_type=jnp.float32) + b2


def _tail_body(B, M, f_ref, info_ref,
               tw_ref, tb_ref, iw_ref, ib_ref,
               a1w_ref, a1b_ref, alw_ref, alb_ref, a2w_ref, a2b_ref,
               o1w_ref, o1b_ref, olw_ref, olb_ref, o2w_ref, o2b_ref,
               feat_ref, pred_ref, off_ref):
    t = jnp.maximum(
        jnp.dot(f_ref[...], tw_ref[...], preferred_element_type=jnp.float32)
        + tb_ref[...], 0.0)
    feats = t.reshape(B, M, 512).max(axis=1)
    feat_ref[...] = feats
    info_f = jnp.dot(info_ref[...], iw_ref[...],
                     preferred_element_type=jnp.float32) + ib_ref[...]
    x = jnp.concatenate([feats, info_f], axis=1)
    pred_ref[...] = _ln_head(x, a1w_ref[...], a1b_ref[...], alw_ref[...],
                             alb_ref[...], a2w_ref[...], a2b_ref[...])
    off_ref[...] = _ln_head(x, o1w_ref[...], o1b_ref[...], olw_ref[...],
                            olb_ref[...], o2w_ref[...], o2b_ref[...])


def _tail_heads(f2d, info, p):
    B = info.shape[0]
    M = f2d.shape[0] // B
    body = functools.partial(_tail_body, B, M)
    full = lambda a: pl.BlockSpec(a.shape, lambda: tuple([0] * a.ndim))
    args = [f2d, info,
            p['tail_W'], p['tail_b'][None], p['info_W'], p['info_b'][None],
            p['a1_W'], p['a1_b'][None], p['a_ln_w'][None], p['a_ln_b'][None],
            p['a2_W'], p['a2_b'][None],
            p['o1_W'], p['o1_b'][None], p['o_ln_w'][None], p['o_ln_b'][None],
            p['o2_W'], p['o2_b'][None]]
    return pl.pallas_call(
        body,
        in_specs=[full(a) for a in args],
        out_specs=[
            pl.BlockSpec((B, 512), lambda: (0, 0)),
            pl.BlockSpec((B, K_CLS), lambda: (0, 0)),
            pl.BlockSpec((B, K_CLS * 3), lambda: (0, 0)),
        ],
        out_shape=[
            jax.ShapeDtypeStruct((B, 512), jnp.float32),
            jax.ShapeDtypeStruct((B, K_CLS), jnp.float32),
            jax.ShapeDtypeStruct((B, K_CLS * 3), jnp.float32),
        ],
    )(*args)


def _bq(centers, xyz, radius, nsample):
    d2 = jnp.sum((centers[:, :, None, :] - xyz[:, None, :, :]) ** 2, axis=-1)
    masked = jnp.where(d2 <= radius * radius, d2, jnp.inf)
    negd, idx = jax.lax.top_k(-masked, nsample)
    first = idx[:, :, :1]
    idx = jnp.where(negd > -jnp.inf, idx, first)
    return idx


def _take(x, idx):
    return jax.vmap(lambda xb, ib: xb[ib])(x, idx)


def kernel(points, info, params):
    p = params
    B, N, _ = points.shape
    xyz = points[..., :3]
    f = _stem(points.reshape(B * N, 4), p['stem_W'], p['stem_b'])
    f = f.reshape(B, N, 32)
    radius = BASE_RADIUS
    for i in range(4):
        new_xyz = xyz[:, ::2]
        f_center = f[:, ::2]
        M = new_xyz.shape[1]
        idx = _bq(new_xyz, xyz, radius, NSAMPLE)
        g_xyz = _take(xyz, idx)
        g_f = _take(f, idx)
        dp = (g_xyz - new_xyz[:, :, None, :]) * (1.0 / radius)
        h = jnp.concatenate([dp, g_f], axis=-1)
        Ci = h.shape[-1]
        g2 = h.reshape(B, M * NSAMPLE, Ci)
        f = _sa_mlp(g2, f_center,
                    p['sa%d_W1' % i], p['sa%d_b1' % i],
                    p['sa%d_W2' % i], p['sa%d_b2' % i],
                    p['sa%d_Wr' % i], p['sa%d_br' % i])
        xyz = new_xyz
        radius = radius * RADIUS_SCALING
    features, pred, off = _tail_heads(f.reshape(B * f.shape[1], 512), info, p)
    return (features, pred, off.reshape(-1, K_CLS, 3))
